# baseline (speedup 1.0000x reference)
import jax
import jax.numpy as jnp
from jax import lax
from jax.experimental import pallas as pl
from jax.experimental.pallas import tpu as pltpu
from jax.experimental.pallas import tpu_sc as plsc

NUM_PROTEINS = 1000000
LATENT_DIM = 32
BATCH = 16384

NC, NS, L = 2, 16, 16
NW = NC * NS
B_PER_W = BATCH // NW
PH_N = 4
PH = B_PER_W // PH_N
WAVE = 32
WAVES = PH // WAVE


def _sc_body(p1_hbm, p2_hbm, emb_hbm, re_hbm, beta_hbm, out_hbm,
             idx1_v, idx2_v, z1p, z2p, rr1p, rr2p, beta_v, out_v, sem):
    wid = lax.axis_index("s") * NC + lax.axis_index("c")
    base = wid * B_PER_W

    pltpu.sync_copy(p1_hbm.at[pl.ds(base, B_PER_W)], idx1_v)
    pltpu.sync_copy(p2_hbm.at[pl.ds(base, B_PER_W)], idx2_v)
    pltpu.sync_copy(beta_hbm, beta_v)

    beta = beta_v[...]
    lane = lax.iota(jnp.int32, L)
    zero = jnp.zeros((L,), jnp.int32)

    def drain_wave():
        pltpu.make_async_copy(
            emb_hbm.at[pl.ds(0, WAVE)],
            z1p.at[pl.ds(0, WAVE), pl.ds(0, LATENT_DIM)], sem).wait()
        pltpu.make_async_copy(
            emb_hbm.at[pl.ds(0, WAVE)],
            z2p.at[pl.ds(0, WAVE), pl.ds(0, LATENT_DIM)], sem).wait()
        pltpu.make_async_copy(
            re_hbm.at[pl.ds(0, WAVE)],
            rr1p.at[pl.ds(0, WAVE), pl.ds(0, 1)], sem).wait()
        pltpu.make_async_copy(
            re_hbm.at[pl.ds(0, WAVE)],
            rr2p.at[pl.ds(0, WAVE), pl.ds(0, 1)], sem).wait()

    for ph in range(PH_N):
        for w in range(WAVES):
            def fire(g, _):
                i = ph * PH + w * WAVE + g * L
                j = w * WAVE + g * L
                v1 = idx1_v[pl.ds(i, L)]
                v2 = idx2_v[pl.ds(i, L)]
                for t in range(L):
                    a1 = v1[t]
                    a2 = v2[t]
                    pltpu.async_copy(
                        emb_hbm.at[pl.ds(a1, 1)],
                        z1p.at[pl.ds(j + t, 1), pl.ds(0, LATENT_DIM)], sem)
                    pltpu.async_copy(
                        emb_hbm.at[pl.ds(a2, 1)],
                        z2p.at[pl.ds(j + t, 1), pl.ds(0, LATENT_DIM)], sem)
                    pltpu.async_copy(
                        re_hbm.at[pl.ds(a1, 1)],
                        rr1p.at[pl.ds(j + t, 1), pl.ds(0, 1)], sem)
                    pltpu.async_copy(
                        re_hbm.at[pl.ds(a2, 1)],
                        rr2p.at[pl.ds(j + t, 1), pl.ds(0, 1)], sem)
                return ()

            lax.fori_loop(0, WAVE // L, fire, ())
            if w > 0:
                drain_wave()
        drain_wave()

        def chunk(c, _):
            rows = c * L + lane
            acc = jnp.zeros((L,), jnp.float32)
            for j in range(LATENT_DIM):
                col = jnp.full((L,), j, jnp.int32)
                a = plsc.load_gather(z1p, [rows, col])
                b = plsc.load_gather(z2p, [rows, col])
                d = a - b
                acc = acc + d * d
            r1 = plsc.load_gather(rr1p, [rows, zero])
            r2 = plsc.load_gather(rr2p, [rows, zero])
            s = jnp.maximum(acc, jnp.float32(1e-35))
            i = lax.bitcast_convert_type(s, jnp.int32)
            i = jnp.int32(0x5F3759DF) - lax.shift_right_arithmetic(i, 1)
            y = lax.bitcast_convert_type(i, jnp.float32)
            for _ in range(3):
                y = y * (jnp.float32(1.5) - jnp.float32(0.5) * s * y * y)
            dist = s * y
            out_v[pl.ds(ph * PH + c * L, L)] = r1 + r2 - beta * dist
            return ()

        lax.fori_loop(0, PH // L, chunk, ())

    pltpu.sync_copy(out_v, out_hbm.at[pl.ds(base, B_PER_W)])


@jax.jit
def _run(p1, p2, emb, re_tab, beta_vec):
    mesh = plsc.VectorSubcoreMesh(core_axis_name="c", subcore_axis_name="s",
                                  num_cores=NC, num_subcores=NS)
    return pl.kernel(
        _sc_body,
        out_type=jax.ShapeDtypeStruct((BATCH,), jnp.float32),
        mesh=mesh,
        compiler_params=pltpu.CompilerParams(needs_layout_passes=False),
        scratch_types=[
            pltpu.VMEM((B_PER_W,), jnp.int32),
            pltpu.VMEM((B_PER_W,), jnp.int32),
            pltpu.VMEM((PH, LATENT_DIM), jnp.float32),
            pltpu.VMEM((PH, LATENT_DIM), jnp.float32),
            pltpu.VMEM((PH, 1), jnp.float32),
            pltpu.VMEM((PH, 1), jnp.float32),
            pltpu.VMEM((L,), jnp.float32),
            pltpu.VMEM((B_PER_W,), jnp.float32),
            pltpu.SemaphoreType.DMA,
        ],
    )(p1, p2, emb, re_tab, beta_vec)


def kernel(protein1_idx, protein2_idx, isoform_embeddings, random_effects, beta_iso):
    beta_vec = jnp.full((L,), beta_iso, jnp.float32)
    return _run(protein1_idx.astype(jnp.int32), protein2_idx.astype(jnp.int32),
                isoform_embeddings, random_effects, beta_vec)


# speedup vs baseline: 1.0483x; 1.0483x over previous
import jax
import jax.numpy as jnp
from jax import lax
from jax.experimental import pallas as pl
from jax.experimental.pallas import tpu as pltpu
from jax.experimental.pallas import tpu_sc as plsc

NUM_PROTEINS = 1000000
LATENT_DIM = 32
BATCH = 16384

NC, NS, L = 2, 16, 16
NW = NC * NS
B_PER_W = BATCH // NW
CHUNKS = B_PER_W // L
IDX_COLS = 128
IDX_ROWS = B_PER_W // IDX_COLS
RE_COLS = 16


def _sc_body(p1_hbm, p2_hbm, emb_hbm, re_hbm, beta_hbm, out_hbm,
             idx1_v, idx2_v, row1_v, row2_v, sub1_v, sub2_v,
             z1_v, z2_v, rr1_v, rr2_v, beta_v, out_v, sem):
    wid = lax.axis_index("s") * NC + lax.axis_index("c")
    base = wid * B_PER_W

    pltpu.sync_copy(p1_hbm.at[pl.ds(wid * IDX_ROWS, IDX_ROWS)], idx1_v)
    pltpu.sync_copy(p2_hbm.at[pl.ds(wid * IDX_ROWS, IDX_ROWS)], idx2_v)

    for k in range(IDX_ROWS):
        for o in range(0, IDX_COLS, L):
            s = pl.ds(o, L)
            v1 = idx1_v[k, s]
            v2 = idx2_v[k, s]
            row1_v[k, s] = lax.shift_right_logical(v1, 4)
            row2_v[k, s] = lax.shift_right_logical(v2, 4)
            d = pl.ds(k * IDX_COLS + o, L)
            sub1_v[d] = lax.bitwise_and(v1, jnp.int32(RE_COLS - 1))
            sub2_v[d] = lax.bitwise_and(v2, jnp.int32(RE_COLS - 1))

    cps = []
    for k in range(IDX_ROWS):
        sl = pl.ds(k * IDX_COLS, IDX_COLS)
        cps += [
            pltpu.async_copy(emb_hbm.at[idx1_v.at[k]], z1_v.at[sl], sem),
            pltpu.async_copy(emb_hbm.at[idx2_v.at[k]], z2_v.at[sl], sem),
            pltpu.async_copy(re_hbm.at[row1_v.at[k]], rr1_v.at[sl], sem),
            pltpu.async_copy(re_hbm.at[row2_v.at[k]], rr2_v.at[sl], sem),
        ]
    pltpu.sync_copy(beta_hbm, beta_v)
    for cp in cps:
        cp.wait()

    beta = beta_v[...]
    lane = lax.iota(jnp.int32, L)

    def chunk(c, _):
        rows = c * L + lane
        acc = jnp.zeros((L,), jnp.float32)
        for j in range(LATENT_DIM):
            col = jnp.full((L,), j, jnp.int32)
            a = plsc.load_gather(z1_v, [rows, col])
            b = plsc.load_gather(z2_v, [rows, col])
            d = a - b
            acc = acc + d * d
        sl = pl.ds(c * L, L)
        r1 = plsc.load_gather(rr1_v, [rows, sub1_v[sl]])
        r2 = plsc.load_gather(rr2_v, [rows, sub2_v[sl]])
        s = jnp.maximum(acc, jnp.float32(1e-35))
        i = lax.bitcast_convert_type(s, jnp.int32)
        i = jnp.int32(0x5F3759DF) - lax.shift_right_arithmetic(i, 1)
        y = lax.bitcast_convert_type(i, jnp.float32)
        for _ in range(3):
            y = y * (jnp.float32(1.5) - jnp.float32(0.5) * s * y * y)
        dist = s * y
        out_v[sl] = r1 + r2 - beta * dist
        return ()

    lax.fori_loop(0, CHUNKS, chunk, ())
    pltpu.sync_copy(out_v, out_hbm.at[pl.ds(base, B_PER_W)])


@jax.jit
def _run(p1, p2, emb, re_tab, beta_vec):
    mesh = plsc.VectorSubcoreMesh(core_axis_name="c", subcore_axis_name="s",
                                  num_cores=NC, num_subcores=NS)
    return pl.kernel(
        _sc_body,
        out_type=jax.ShapeDtypeStruct((BATCH,), jnp.float32),
        mesh=mesh,
        compiler_params=pltpu.CompilerParams(needs_layout_passes=False,
                                             use_tc_tiling_on_sc=False),
        scratch_types=[
            pltpu.VMEM((IDX_ROWS, IDX_COLS), jnp.int32),
            pltpu.VMEM((IDX_ROWS, IDX_COLS), jnp.int32),
            pltpu.VMEM((IDX_ROWS, IDX_COLS), jnp.int32),
            pltpu.VMEM((IDX_ROWS, IDX_COLS), jnp.int32),
            pltpu.VMEM((B_PER_W,), jnp.int32),
            pltpu.VMEM((B_PER_W,), jnp.int32),
            pltpu.VMEM((B_PER_W, LATENT_DIM), jnp.float32),
            pltpu.VMEM((B_PER_W, LATENT_DIM), jnp.float32),
            pltpu.VMEM((B_PER_W, RE_COLS), jnp.float32),
            pltpu.VMEM((B_PER_W, RE_COLS), jnp.float32),
            pltpu.VMEM((L,), jnp.float32),
            pltpu.VMEM((B_PER_W,), jnp.float32),
            pltpu.SemaphoreType.DMA,
        ],
    )(p1, p2, emb, re_tab, beta_vec)


def kernel(protein1_idx, protein2_idx, isoform_embeddings, random_effects, beta_iso):
    beta_vec = jnp.full((L,), beta_iso, jnp.float32)
    p1 = protein1_idx.astype(jnp.int32).reshape(BATCH // IDX_COLS, IDX_COLS)
    p2 = protein2_idx.astype(jnp.int32).reshape(BATCH // IDX_COLS, IDX_COLS)
    re2 = random_effects.reshape(NUM_PROTEINS // RE_COLS, RE_COLS)
    return _run(p1, p2, isoform_embeddings, re2, beta_vec)
